# baseline (speedup 1.0000x reference)
import functools

import jax
import jax.numpy as jnp
from jax import lax
from jax.experimental import pallas as pl
from jax.experimental.pallas import tpu as pltpu
from jax.experimental.pallas import tpu_sc as plsc

def _take16(v, idx):
    dnums = lax.GatherDimensionNumbers(
        offset_dims=(), collapsed_slice_dims=(0,), start_index_map=(0,))
    return lax.gather(v, idx[:, None], dnums, slice_sizes=(1,),
                      mode=lax.GatherScatterMode.PROMISE_IN_BOUNDS)


N = 100000
D = 128
S = 256
L = 16
KV = D // L
NC = 2
NS = 16
NW = NC * NS
RPW = N // NW
CH = 25
CHP = 32
NCH = RPW // CH


def _sc_body(x_hbm, ids_hbm, w_hbm, zacc_hbm, zcnt_hbm, ones_hbm,
             oacc_hbm, ocnt_hbm,
             buf, idxv, onesv, wv, acc, cnt):
    c = lax.axis_index("c")
    s = lax.axis_index("s")
    wid = c * NS + s

    pltpu.sync_copy(ones_hbm, onesv)
    zero = jnp.zeros((L,), jnp.float32)
    for j in range(CH, CHP):
        for k in range(KV):
            buf[j, k] = zero

    lanes = lax.iota(jnp.int32, L)
    perms = [lanes ^ m for m in (8, 4, 2, 1)]

    row0 = wid * RPW
    blk0 = wid * NCH
    for p in range(3):
        @pl.when(s == 0)
        def _init(p=p):
            pltpu.sync_copy(zacc_hbm, acc)
            pltpu.sync_copy(zcnt_hbm, cnt)
            pltpu.sync_copy(w_hbm.at[p], wv)

        plsc.subcore_barrier()
        wk = [wv[k] for k in range(KV)]
        bvec = wv[KV]

        def chunk_body(ch, carry, p=p, wk=wk, bvec=bvec):
            r0 = row0 + ch * CH
            pltpu.sync_copy(x_hbm.at[p, pl.ds(r0, CH)], buf.at[pl.ds(0, CH)])
            pltpu.sync_copy(ids_hbm.at[p, blk0 + ch], idxv)

            def row_fn(r, carry2):
                xs = [buf[r, k] for k in range(KV)]
                dot = xs[0] * wk[0]
                for k in range(1, KV):
                    dot = dot + xs[k] * wk[k]
                for pm in perms:
                    dot = dot + _take16(dot, pm)
                z = dot + bvec
                a = 1.0 / (1.0 + jnp.exp(-z))
                for k in range(KV):
                    buf[r, k] = xs[k] * a
                return carry2

            lax.fori_loop(0, CH, row_fn, 0)
            pltpu.sync_copy(buf, acc.at[idxv], add=True)
            pltpu.sync_copy(onesv, cnt.at[idxv], add=True)
            return carry

        lax.fori_loop(0, NCH, chunk_body, 0)
        plsc.subcore_barrier()

        @pl.when(s == 0)
        def _out(p=p):
            pltpu.sync_copy(acc, oacc_hbm.at[c, p])
            pltpu.sync_copy(cnt, ocnt_hbm.at[c, p])

        plsc.subcore_barrier()


@jax.jit
def _sc_call(x_all, ids_pad, w_all, zacc, zcnt, ones):
    mesh = plsc.VectorSubcoreMesh(core_axis_name="c", subcore_axis_name="s")
    f = pl.kernel(
        _sc_body,
        mesh=mesh,
        out_type=[
            jax.ShapeDtypeStruct((NC, 3, S, KV, L), jnp.float32),
            jax.ShapeDtypeStruct((NC, 3, S, L), jnp.float32),
        ],
        scratch_types=[
            pltpu.VMEM((CHP, KV, L), jnp.float32),
            pltpu.VMEM((CHP,), jnp.int32),
            pltpu.VMEM((CHP, L), jnp.float32),
            pltpu.VMEM((KV + 2, L), jnp.float32),
            pltpu.VMEM_SHARED((S, KV, L), jnp.float32),
            pltpu.VMEM_SHARED((S, L), jnp.float32),
        ],
    )
    return f(x_all, ids_pad, w_all, zacc, zcnt, ones)


def _fin_body(pa_ref, pc_ref, nw_ref, nb_ref, o_ref):
    acc = pa_ref[0] + pa_ref[1]
    cnt = pc_ref[0] + pc_ref[1]
    c1 = cnt[:, :, 0:1]
    e = acc / jnp.maximum(c1, 1.0)
    ecat = jnp.concatenate([e[0], e[1], e[2]], axis=1)
    o_ref[...] = (
        jnp.dot(ecat, nw_ref[...], preferred_element_type=jnp.float32)
        + nb_ref[...]
    )


@jax.jit
def _fin_call(pacc, pcnt, net_w, net_b):
    return pl.pallas_call(
        _fin_body,
        out_shape=jax.ShapeDtypeStruct((S, D), jnp.float32),
    )(pacc, pcnt, net_w, net_b)


def kernel(x_u, x_v, x_y, index_u, index_v, index_y,
           att_w_u, att_b_u, att_w_v, att_b_v, att_w_y, att_b_y,
           net_w, net_b):
    x_all = jnp.stack([x_u, x_v, x_y]).reshape(3, N, KV, L)
    ids = jnp.stack([index_u, index_v, index_y]).astype(jnp.int32)
    ids_pad = jnp.pad(ids.reshape(3, NW * NCH, CH),
                      ((0, 0), (0, 0), (0, CHP - CH)))
    w_all = jnp.stack([att_w_u, att_w_v, att_w_y]).reshape(3, KV, L)
    b_bcast = jnp.broadcast_to(
        jnp.stack([att_b_u, att_b_v, att_b_y]).reshape(3, 1, 1), (3, 1, L))
    w_all = jnp.concatenate(
        [w_all, b_bcast, jnp.zeros((3, 1, L), jnp.float32)], axis=1)
    zacc = jnp.zeros((S, KV, L), jnp.float32)
    zcnt = jnp.zeros((S, L), jnp.float32)
    ones = jnp.concatenate(
        [jnp.ones((CH, L), jnp.float32),
         jnp.zeros((CHP - CH, L), jnp.float32)])
    oacc, ocnt = _sc_call(x_all, ids_pad, w_all, zacc, zcnt, ones)
    return _fin_call(oacc.reshape(NC, 3, S, D), ocnt, net_w,
                     net_b.reshape(1, D))
